# baseline (speedup 1.0000x reference)
import functools

import jax
import jax.numpy as jnp
from jax import lax
from jax.experimental import pallas as pl
from jax.experimental.pallas import tpu as pltpu
from jax.experimental.pallas import tpu_sc as plsc

LANES = 16
NCORES = 2
NSUB = 16
NW = NCORES * NSUB


def _wid():
  return lax.axis_index("s") * NCORES + lax.axis_index("c")


def _make_deg_kernel(NP, E):
  EPW = E // NW
  mesh = plsc.VectorSubcoreMesh(core_axis_name="c", subcore_axis_name="s",
                                num_cores=NCORES, num_subcores=NSUB)

  @functools.partial(
      pl.kernel,
      out_type=jax.ShapeDtypeStruct((NW, NP), jnp.float32),
      mesh=mesh,
      scratch_types=[
          pltpu.VMEM((EPW,), jnp.int32),
          pltpu.VMEM((EPW,), jnp.float32),
          pltpu.VMEM((NP,), jnp.float32),
      ],
      compiler_params=pltpu.CompilerParams(needs_layout_passes=False),
  )
  def deg_k(dst_hbm, ew_hbm, out_hbm, dst_v, ew_v, acc_v):
    wid = _wid()
    base = wid * EPW
    pltpu.sync_copy(dst_hbm.at[pl.ds(base, EPW)], dst_v)
    pltpu.sync_copy(ew_hbm.at[pl.ds(base, EPW)], ew_v)

    @plsc.parallel_loop(0, NP // LANES, unroll=8)
    def _(i):
      acc_v[pl.ds(i * LANES, LANES)] = jnp.zeros((LANES,), jnp.float32)

    @plsc.parallel_loop(0, EPW // LANES, unroll=8)
    def _(g):
      dv = dst_v[pl.ds(g * LANES, LANES)]
      wv = ew_v[pl.ds(g * LANES, LANES)]
      plsc.addupdate_scatter(acc_v, [dv], wv)

    pltpu.sync_copy(acc_v, out_hbm.at[wid])

  return deg_k


def _make_agg_kernel(NP, E, F, CH, SH):
  mesh = plsc.VectorSubcoreMesh(core_axis_name="c", subcore_axis_name="s",
                                num_cores=NCORES, num_subcores=NSUB)
  NCH = E // CH
  assert NCH % 2 == 0 and CH % LANES == 0
  scratch = (
      [pltpu.VMEM((NP,), jnp.float32) for _ in range(F)]
      + [pltpu.VMEM((NP,), jnp.float32) for _ in range(F)]
      + [pltpu.VMEM((2, CH), jnp.int32),
         pltpu.VMEM((2, CH), jnp.float32),
         pltpu.VMEM((NP,), jnp.float32),
         pltpu.VMEM((NW * F,), jnp.float32),
         pltpu.SemaphoreType.DMA,
         pltpu.SemaphoreType.DMA]
  )

  @functools.partial(
      pl.kernel,
      out_type=jax.ShapeDtypeStruct((NW * F, NP), jnp.float32),
      mesh=mesh,
      scratch_types=scratch,
      compiler_params=pltpu.CompilerParams(needs_layout_passes=False),
  )
  def agg_k(y_hbm, pk_hbm, ew_hbm, dinv_hbm, b_hbm, out_hbm, *refs):
    xf = refs[0:F]
    acc = refs[F:2 * F]
    pk_v, ew_v = refs[2 * F:2 * F + 2]
    dinv_v, b_v = refs[2 * F + 2:2 * F + 4]
    sems = refs[2 * F + 4:2 * F + 6]
    wid = _wid()
    fbase = wid * F
    pre = [pltpu.async_copy(y_hbm.at[fbase + f], xf[f], sems[0])
           for f in range(F)]
    pre.append(pltpu.async_copy(dinv_hbm, dinv_v, sems[0]))
    pre.append(pltpu.async_copy(b_hbm, b_v, sems[0]))

    @plsc.parallel_loop(0, NP // LANES, unroll=8)
    def _(i):
      for f in range(F):
        acc[f][pl.ds(i * LANES, LANES)] = jnp.zeros((LANES,), jnp.float32)

    for cp in pre:
      cp.wait()

    def issue(c, b):
      base = c * CH
      pltpu.async_copy(pk_hbm.at[pl.ds(base, CH)], pk_v.at[b], sems[b])
      pltpu.async_copy(ew_hbm.at[pl.ds(base, CH)], ew_v.at[b], sems[b])

    def drain(c, b):
      base = c * CH
      pltpu.make_async_copy(pk_hbm.at[pl.ds(base, CH)], pk_v.at[b],
                            sems[b]).wait()
      pltpu.make_async_copy(ew_hbm.at[pl.ds(base, CH)], ew_v.at[b],
                            sems[b]).wait()

    def process(b):
      @plsc.parallel_loop(0, CH // LANES, unroll=8)
      def _(g):
        pv = pk_v[b, pl.ds(g * LANES, LANES)]
        wv = ew_v[b, pl.ds(g * LANES, LANES)]
        sv = pv & ((1 << SH) - 1)
        dv = lax.shift_right_logical(pv, SH)
        for f in range(F):
          gv = plsc.load_gather(xf[f], [sv])
          plsc.addupdate_scatter(acc[f], [dv], gv * wv)

    issue(0, 0)

    def chunk_pair(c2, carry):
      c = 2 * c2
      issue(c + 1, 1)
      drain(c, 0)
      process(0)

      @pl.when(c2 < NCH // 2 - 1)
      def _():
        issue(c + 2, 0)

      drain(c + 1, 1)
      process(1)
      return carry

    lax.fori_loop(0, NCH // 2, chunk_pair, 0)

    bspl = [plsc.load_gather(b_v, [jnp.zeros((LANES,), jnp.int32) + (fbase + f)])
            for f in range(F)]

    @plsc.parallel_loop(0, NP // LANES, unroll=8)
    def _(i):
      sl = pl.ds(i * LANES, LANES)
      dv = dinv_v[sl]
      for f in range(F):
        acc[f][sl] = dv * (acc[f][sl] + xf[f][sl]) + bspl[f]

    post = [pltpu.async_copy(acc[f], out_hbm.at[fbase + f], sems[0])
            for f in range(F)]
    for cp in post:
      cp.wait()

  return agg_k


def _make_tc_mm1(NP, D, H, BN):
  def body(xT_ref, w1_ref, xw_ref):
    xw_ref[...] = lax.dot_general(w1_ref[...], xT_ref[...],
                                  (((0,), (0,)), ((), ())),
                                  preferred_element_type=jnp.float32)

  return pl.pallas_call(
      body,
      grid=(NP // BN,),
      in_specs=[
          pl.BlockSpec((D, BN), lambda j: (0, j)),
          pl.BlockSpec((D, H), lambda j: (0, 0)),
      ],
      out_specs=pl.BlockSpec((H, BN), lambda j: (0, j)),
      out_shape=jax.ShapeDtypeStruct((H, NP), jnp.float32),
  )


def _make_tc_scale(NP, H, BN):
  def body(parts_ref, xw_ref, y1_ref, dinv_ref):
    deg = 1.0 + jnp.sum(parts_ref[...], axis=0)
    dinv = jnp.where(deg > 0, lax.rsqrt(jnp.maximum(deg, 1e-12)), 0.0)
    y1_ref[...] = xw_ref[...] * dinv[None, :]
    dinv_ref[...] = dinv[None, :]

  return pl.pallas_call(
      body,
      grid=(NP // BN,),
      in_specs=[
          pl.BlockSpec((NW, BN), lambda j: (0, j)),
          pl.BlockSpec((H, BN), lambda j: (0, j)),
      ],
      out_specs=[
          pl.BlockSpec((H, BN), lambda j: (0, j)),
          pl.BlockSpec((1, BN), lambda j: (0, j)),
      ],
      out_shape=[
          jax.ShapeDtypeStruct((H, NP), jnp.float32),
          jax.ShapeDtypeStruct((1, NP), jnp.float32),
      ],
  )


def _make_tc_mid(NP, H, O, BN):
  def body(h_ref, dinv_ref, w2_ref, y2_ref):
    xw2 = lax.dot_general(w2_ref[...], h_ref[...],
                          (((0,), (0,)), ((), ())),
                          preferred_element_type=jnp.float32)
    y2_ref[...] = xw2 * dinv_ref[...]

  return pl.pallas_call(
      body,
      grid=(NP // BN,),
      in_specs=[
          pl.BlockSpec((H, BN), lambda j: (0, j)),
          pl.BlockSpec((1, BN), lambda j: (0, j)),
          pl.BlockSpec((H, O), lambda j: (0, 0)),
      ],
      out_specs=pl.BlockSpec((O, BN), lambda j: (0, j)),
      out_shape=jax.ShapeDtypeStruct((O, NP), jnp.float32),
  )


def kernel(x, edge_index, edge_weight, W1, b1, W2, b2):
  N, D = x.shape
  H = W1.shape[1]
  O = W2.shape[1]
  E = edge_weight.shape[0]

  BN = 512
  NP = ((N + BN - 1) // BN) * BN
  F = H // NW
  CH = 6400
  SH = max(int(N - 1).bit_length(), 1)
  assert 2 * SH <= 31

  src = edge_index[0]
  dst = edge_index[1]
  xT = jnp.pad(x, ((0, NP - N), (0, 0))).T
  packed = jnp.bitwise_or(jnp.left_shift(dst, SH), src)

  deg_k = _make_deg_kernel(NP, E)
  agg_k = _make_agg_kernel(NP, E, F, CH, SH)
  tc_mm1 = _make_tc_mm1(NP, D, H, BN)
  tc_scale = _make_tc_scale(NP, H, BN)
  tc_mid = _make_tc_mid(NP, H, O, BN)

  parts = deg_k(dst, edge_weight)
  xw1 = tc_mm1(xT, W1)
  y1, dinv = tc_scale(parts, xw1)
  dinv_flat = dinv.reshape(NP)
  h = agg_k(y1, packed, edge_weight, dinv_flat, b1)
  y2 = tc_mid(h, dinv, W2)
  outT = agg_k(y2, packed, edge_weight, dinv_flat, b2)
  return outT[:, :N].T


# speedup vs baseline: 1.0246x; 1.0246x over previous
import functools

import jax
import jax.numpy as jnp
from jax import lax
from jax.experimental import pallas as pl
from jax.experimental.pallas import tpu as pltpu
from jax.experimental.pallas import tpu_sc as plsc

LANES = 16
NCORES = 2
NSUB = 16
NW = NCORES * NSUB


def _wid():
  return lax.axis_index("s") * NCORES + lax.axis_index("c")


def _make_deg_kernel(NP, E):
  EPW = E // NW
  mesh = plsc.VectorSubcoreMesh(core_axis_name="c", subcore_axis_name="s",
                                num_cores=NCORES, num_subcores=NSUB)

  @functools.partial(
      pl.kernel,
      out_type=jax.ShapeDtypeStruct((NW, NP), jnp.float32),
      mesh=mesh,
      scratch_types=[
          pltpu.VMEM((EPW,), jnp.int32),
          pltpu.VMEM((EPW,), jnp.float32),
          pltpu.VMEM((NP,), jnp.float32),
      ],
      compiler_params=pltpu.CompilerParams(needs_layout_passes=False),
  )
  def deg_k(dst_hbm, ew_hbm, out_hbm, dst_v, ew_v, acc_v):
    wid = _wid()
    base = wid * EPW
    pltpu.sync_copy(dst_hbm.at[pl.ds(base, EPW)], dst_v)
    pltpu.sync_copy(ew_hbm.at[pl.ds(base, EPW)], ew_v)

    @plsc.parallel_loop(0, NP // LANES, unroll=8)
    def _(i):
      acc_v[pl.ds(i * LANES, LANES)] = jnp.zeros((LANES,), jnp.float32)

    @plsc.parallel_loop(0, EPW // LANES, unroll=8)
    def _(g):
      dv = dst_v[pl.ds(g * LANES, LANES)]
      wv = ew_v[pl.ds(g * LANES, LANES)]
      plsc.addupdate_scatter(acc_v, [dv], wv)

    pltpu.sync_copy(acc_v, out_hbm.at[wid])

  return deg_k


def _make_agg_kernel(NP, E, F, CH, SH):
  mesh = plsc.VectorSubcoreMesh(core_axis_name="c", subcore_axis_name="s",
                                num_cores=NCORES, num_subcores=NSUB)
  NCH = E // CH
  assert NCH % 2 == 0 and CH % LANES == 0
  scratch = (
      [pltpu.VMEM((NP,), jnp.float32) for _ in range(F)]
      + [pltpu.VMEM((NP,), jnp.float32) for _ in range(F)]
      + [pltpu.VMEM((2, CH), jnp.int32),
         pltpu.VMEM((2, CH), jnp.float32),
         pltpu.VMEM((NP,), jnp.float32),
         pltpu.VMEM((NW * F,), jnp.float32),
         pltpu.SemaphoreType.DMA,
         pltpu.SemaphoreType.DMA]
  )

  @functools.partial(
      pl.kernel,
      out_type=jax.ShapeDtypeStruct((NW * F, NP), jnp.float32),
      mesh=mesh,
      scratch_types=scratch,
      compiler_params=pltpu.CompilerParams(needs_layout_passes=False),
  )
  def agg_k(y_hbm, pk_hbm, ew_hbm, dinv_hbm, b_hbm, out_hbm, *refs):
    xf = refs[0:F]
    acc = refs[F:2 * F]
    pk_v, ew_v = refs[2 * F:2 * F + 2]
    dinv_v, b_v = refs[2 * F + 2:2 * F + 4]
    sems = refs[2 * F + 4:2 * F + 6]
    wid = _wid()
    fbase = wid * F
    pre = [pltpu.async_copy(y_hbm.at[fbase + f], xf[f], sems[0])
           for f in range(F)]
    pre.append(pltpu.async_copy(dinv_hbm, dinv_v, sems[0]))
    pre.append(pltpu.async_copy(b_hbm, b_v, sems[0]))

    @plsc.parallel_loop(0, NP // LANES, unroll=8)
    def _(i):
      for f in range(F):
        acc[f][pl.ds(i * LANES, LANES)] = jnp.zeros((LANES,), jnp.float32)

    for cp in pre:
      cp.wait()

    def issue(c, b):
      base = c * CH
      pltpu.async_copy(pk_hbm.at[pl.ds(base, CH)], pk_v.at[b], sems[b])
      pltpu.async_copy(ew_hbm.at[pl.ds(base, CH)], ew_v.at[b], sems[b])

    def drain(c, b):
      base = c * CH
      pltpu.make_async_copy(pk_hbm.at[pl.ds(base, CH)], pk_v.at[b],
                            sems[b]).wait()
      pltpu.make_async_copy(ew_hbm.at[pl.ds(base, CH)], ew_v.at[b],
                            sems[b]).wait()

    def process(b):
      @plsc.parallel_loop(0, CH // LANES, unroll=8)
      def _(g):
        pv = pk_v[b, pl.ds(g * LANES, LANES)]
        wv = ew_v[b, pl.ds(g * LANES, LANES)]
        sv = pv & ((1 << SH) - 1)
        dv = lax.shift_right_logical(pv, SH)
        for f in range(F):
          gv = plsc.load_gather(xf[f], [sv])
          plsc.addupdate_scatter(acc[f], [dv], gv * wv)

    issue(0, 0)

    def chunk_pair(c2, carry):
      c = 2 * c2
      issue(c + 1, 1)
      drain(c, 0)
      process(0)

      @pl.when(c2 < NCH // 2 - 1)
      def _():
        issue(c + 2, 0)

      drain(c + 1, 1)
      process(1)
      return carry

    lax.fori_loop(0, NCH // 2, chunk_pair, 0)

    bspl = [plsc.load_gather(b_v, [jnp.zeros((LANES,), jnp.int32) + (fbase + f)])
            for f in range(F)]

    @plsc.parallel_loop(0, NP // LANES, unroll=8)
    def _(i):
      sl = pl.ds(i * LANES, LANES)
      dv = dinv_v[sl]
      for f in range(F):
        acc[f][sl] = dv * (acc[f][sl] + xf[f][sl]) + bspl[f]

    post = [pltpu.async_copy(acc[f], out_hbm.at[fbase + f], sems[0])
            for f in range(F)]
    for cp in post:
      cp.wait()

  return agg_k


def _make_tc_prep(NP, D, H, BN):
  def body(parts_ref, xT_ref, w1_ref, y1_ref, dinv_ref):
    deg = 1.0 + jnp.sum(parts_ref[...], axis=0)
    dinv = jnp.where(deg > 0, lax.rsqrt(jnp.maximum(deg, 1e-12)), 0.0)
    xw = lax.dot_general(w1_ref[...], xT_ref[...],
                         (((0,), (0,)), ((), ())),
                         preferred_element_type=jnp.float32)
    y1_ref[...] = xw * dinv[None, :]
    dinv_ref[...] = dinv[None, :]

  return pl.pallas_call(
      body,
      grid=(NP // BN,),
      in_specs=[
          pl.BlockSpec((NW, BN), lambda j: (0, j)),
          pl.BlockSpec((D, BN), lambda j: (0, j)),
          pl.BlockSpec((D, H), lambda j: (0, 0)),
      ],
      out_specs=[
          pl.BlockSpec((H, BN), lambda j: (0, j)),
          pl.BlockSpec((1, BN), lambda j: (0, j)),
      ],
      out_shape=[
          jax.ShapeDtypeStruct((H, NP), jnp.float32),
          jax.ShapeDtypeStruct((1, NP), jnp.float32),
      ],
  )


def _make_tc_mid(NP, H, O, BN):
  def body(h_ref, dinv_ref, w2_ref, y2_ref):
    xw2 = lax.dot_general(w2_ref[...], h_ref[...],
                          (((0,), (0,)), ((), ())),
                          preferred_element_type=jnp.float32)
    y2_ref[...] = xw2 * dinv_ref[...]

  return pl.pallas_call(
      body,
      grid=(NP // BN,),
      in_specs=[
          pl.BlockSpec((H, BN), lambda j: (0, j)),
          pl.BlockSpec((1, BN), lambda j: (0, j)),
          pl.BlockSpec((H, O), lambda j: (0, 0)),
      ],
      out_specs=pl.BlockSpec((O, BN), lambda j: (0, j)),
      out_shape=jax.ShapeDtypeStruct((O, NP), jnp.float32),
  )


def kernel(x, edge_index, edge_weight, W1, b1, W2, b2):
  N, D = x.shape
  H = W1.shape[1]
  O = W2.shape[1]
  E = edge_weight.shape[0]

  BN = 512
  NP = ((N + BN - 1) // BN) * BN
  F = H // NW
  CH = 6400
  SH = max(int(N - 1).bit_length(), 1)
  assert 2 * SH <= 31

  src = edge_index[0]
  dst = edge_index[1]
  xT = jnp.pad(x, ((0, NP - N), (0, 0))).T
  packed = jnp.bitwise_or(jnp.left_shift(dst, SH), src)

  deg_k = _make_deg_kernel(NP, E)
  agg_k = _make_agg_kernel(NP, E, F, CH, SH)
  tc_prep = _make_tc_prep(NP, D, H, BN)
  tc_mid = _make_tc_mid(NP, H, O, BN)

  parts = deg_k(dst, edge_weight)
  y1, dinv = tc_prep(parts, xT, W1)
  dinv_flat = dinv.reshape(NP)
  h = agg_k(y1, packed, edge_weight, dinv_flat, b1)
  y2 = tc_mid(h, dinv, W2)
  outT = agg_k(y2, packed, edge_weight, dinv_flat, b2)
  return outT[:, :N].T


# speedup vs baseline: 1.1214x; 1.0945x over previous
import functools

import jax
import jax.numpy as jnp
from jax import lax
from jax.experimental import pallas as pl
from jax.experimental.pallas import tpu as pltpu
from jax.experimental.pallas import tpu_sc as plsc

LANES = 16
NCORES = 2
NSUB = 16
NW = NCORES * NSUB


def _wid():
  return lax.axis_index("s") * NCORES + lax.axis_index("c")


def _make_deg_kernel(NP, E):
  EPW = E // NW
  mesh = plsc.VectorSubcoreMesh(core_axis_name="c", subcore_axis_name="s",
                                num_cores=NCORES, num_subcores=NSUB)

  @functools.partial(
      pl.kernel,
      out_type=jax.ShapeDtypeStruct((NW, NP), jnp.float32),
      mesh=mesh,
      scratch_types=[
          pltpu.VMEM((EPW,), jnp.int32),
          pltpu.VMEM((EPW,), jnp.float32),
          pltpu.VMEM((NP,), jnp.float32),
      ],
      compiler_params=pltpu.CompilerParams(needs_layout_passes=False),
  )
  def deg_k(dst_hbm, ew_hbm, out_hbm, dst_v, ew_v, acc_v):
    wid = _wid()
    base = wid * EPW
    pltpu.sync_copy(dst_hbm.at[pl.ds(base, EPW)], dst_v)
    pltpu.sync_copy(ew_hbm.at[pl.ds(base, EPW)], ew_v)

    @plsc.parallel_loop(0, NP // LANES, unroll=8)
    def _(i):
      acc_v[pl.ds(i * LANES, LANES)] = jnp.zeros((LANES,), jnp.float32)

    @plsc.parallel_loop(0, EPW // LANES, unroll=8)
    def _(g):
      dv = dst_v[pl.ds(g * LANES, LANES)]
      wv = ew_v[pl.ds(g * LANES, LANES)]
      plsc.addupdate_scatter(acc_v, [dv], wv)

    pltpu.sync_copy(acc_v, out_hbm.at[wid])

  return deg_k


def _make_agg_kernel(NP, E, F, CH, SH):
  mesh = plsc.VectorSubcoreMesh(core_axis_name="c", subcore_axis_name="s",
                                num_cores=NCORES, num_subcores=NSUB)
  NCH = E // CH
  FP = F // 2
  assert NCH % 2 == 0 and CH % LANES == 0 and F % 2 == 0
  scratch = (
      [pltpu.VMEM((NP,), jnp.float32) for _ in range(2)]
      + [pltpu.VMEM((NP,), jnp.int32) for _ in range(FP)]
      + [pltpu.VMEM((NP,), jnp.float32) for _ in range(F)]
      + [pltpu.VMEM((2, CH), jnp.int32),
         pltpu.VMEM((2, CH), jnp.float32),
         pltpu.VMEM((NP,), jnp.float32),
         pltpu.VMEM((NW * F,), jnp.float32),
         pltpu.SemaphoreType.DMA,
         pltpu.SemaphoreType.DMA]
  )

  @functools.partial(
      pl.kernel,
      out_type=jax.ShapeDtypeStruct((NW * F, NP), jnp.float32),
      mesh=mesh,
      scratch_types=scratch,
      compiler_params=pltpu.CompilerParams(needs_layout_passes=False),
  )
  def agg_k(y_hbm, pk_hbm, ew_hbm, dinv_hbm, b_hbm, out_hbm, *refs):
    ta, tb = refs[0:2]
    xfp = refs[2:2 + FP]
    acc = refs[2 + FP:2 + FP + F]
    pk_v, ew_v = refs[2 + FP + F:2 + FP + F + 2]
    dinv_v, b_v = refs[2 + FP + F + 2:2 + FP + F + 4]
    sems = refs[2 + FP + F + 4:2 + FP + F + 6]
    wid = _wid()
    fbase = wid * F
    cp_d = pltpu.async_copy(dinv_hbm, dinv_v, sems[1])
    cp_b = pltpu.async_copy(b_hbm, b_v, sems[1])
    for p in range(FP):
      ca = pltpu.async_copy(y_hbm.at[fbase + 2 * p], ta, sems[0])
      cb = pltpu.async_copy(y_hbm.at[fbase + 2 * p + 1], tb, sems[0])
      ca.wait()
      cb.wait()

      @plsc.parallel_loop(0, NP // LANES, unroll=8)
      def _(i, p=p):
        sl = pl.ds(i * LANES, LANES)
        pair = plsc.pack(ta[sl], tb[sl], format=plsc.PackFormat.INTERLEAVED,
                         preferred_element_type=jnp.bfloat16)
        xfp[p][sl] = plsc.bitcast(pair, jnp.int32)

    @plsc.parallel_loop(0, NP // LANES, unroll=8)
    def _(i):
      for f in range(F):
        acc[f][pl.ds(i * LANES, LANES)] = jnp.zeros((LANES,), jnp.float32)

    cp_d.wait()
    cp_b.wait()

    def issue(c, b):
      base = c * CH
      pltpu.async_copy(pk_hbm.at[pl.ds(base, CH)], pk_v.at[b], sems[b])
      pltpu.async_copy(ew_hbm.at[pl.ds(base, CH)], ew_v.at[b], sems[b])

    def drain(c, b):
      base = c * CH
      pltpu.make_async_copy(pk_hbm.at[pl.ds(base, CH)], pk_v.at[b],
                            sems[b]).wait()
      pltpu.make_async_copy(ew_hbm.at[pl.ds(base, CH)], ew_v.at[b],
                            sems[b]).wait()

    def process(b):
      @plsc.parallel_loop(0, CH // LANES, unroll=8)
      def _(g):
        pv = pk_v[b, pl.ds(g * LANES, LANES)]
        wv = ew_v[b, pl.ds(g * LANES, LANES)]
        sv = pv & ((1 << SH) - 1)
        dv = lax.shift_right_logical(pv, SH)
        for p in range(FP):
          gp = plsc.load_gather(xfp[p], [sv])
          ga, gb = plsc.unpack(plsc.bitcast(gp, jnp.bfloat16),
                               format=plsc.PackFormat.INTERLEAVED,
                               preferred_element_type=jnp.float32)
          plsc.addupdate_scatter(acc[2 * p], [dv], ga * wv)
          plsc.addupdate_scatter(acc[2 * p + 1], [dv], gb * wv)

    issue(0, 0)

    def chunk_pair(c2, carry):
      c = 2 * c2
      issue(c + 1, 1)
      drain(c, 0)
      process(0)

      @pl.when(c2 < NCH // 2 - 1)
      def _():
        issue(c + 2, 0)

      drain(c + 1, 1)
      process(1)
      return carry

    lax.fori_loop(0, NCH // 2, chunk_pair, 0)

    bspl = [plsc.load_gather(b_v, [jnp.zeros((LANES,), jnp.int32) + (fbase + f)])
            for f in range(F)]

    @plsc.parallel_loop(0, NP // LANES, unroll=8)
    def _(i):
      sl = pl.ds(i * LANES, LANES)
      dv = dinv_v[sl]
      for p in range(FP):
        ya, yb = plsc.unpack(plsc.bitcast(xfp[p][sl], jnp.bfloat16),
                             format=plsc.PackFormat.INTERLEAVED,
                             preferred_element_type=jnp.float32)
        acc[2 * p][sl] = dv * (acc[2 * p][sl] + ya) + bspl[2 * p]
        acc[2 * p + 1][sl] = dv * (acc[2 * p + 1][sl] + yb) + bspl[2 * p + 1]

    post = [pltpu.async_copy(acc[f], out_hbm.at[fbase + f], sems[0])
            for f in range(F)]
    for cp in post:
      cp.wait()

  return agg_k


def _make_tc_prep(NP, D, H, BN):
  def body(parts_ref, xT_ref, w1_ref, y1_ref, dinv_ref):
    deg = 1.0 + jnp.sum(parts_ref[...], axis=0)
    dinv = jnp.where(deg > 0, lax.rsqrt(jnp.maximum(deg, 1e-12)), 0.0)
    xw = lax.dot_general(w1_ref[...], xT_ref[...],
                         (((0,), (0,)), ((), ())),
                         preferred_element_type=jnp.float32)
    y1_ref[...] = xw * dinv[None, :]
    dinv_ref[...] = dinv[None, :]

  return pl.pallas_call(
      body,
      grid=(NP // BN,),
      in_specs=[
          pl.BlockSpec((NW, BN), lambda j: (0, j)),
          pl.BlockSpec((D, BN), lambda j: (0, j)),
          pl.BlockSpec((D, H), lambda j: (0, 0)),
      ],
      out_specs=[
          pl.BlockSpec((H, BN), lambda j: (0, j)),
          pl.BlockSpec((1, BN), lambda j: (0, j)),
      ],
      out_shape=[
          jax.ShapeDtypeStruct((H, NP), jnp.float32),
          jax.ShapeDtypeStruct((1, NP), jnp.float32),
      ],
  )


def _make_tc_mid(NP, H, O, BN):
  def body(h_ref, dinv_ref, w2_ref, y2_ref):
    xw2 = lax.dot_general(w2_ref[...], h_ref[...],
                          (((0,), (0,)), ((), ())),
                          preferred_element_type=jnp.float32)
    y2_ref[...] = xw2 * dinv_ref[...]

  return pl.pallas_call(
      body,
      grid=(NP // BN,),
      in_specs=[
          pl.BlockSpec((H, BN), lambda j: (0, j)),
          pl.BlockSpec((1, BN), lambda j: (0, j)),
          pl.BlockSpec((H, O), lambda j: (0, 0)),
      ],
      out_specs=pl.BlockSpec((O, BN), lambda j: (0, j)),
      out_shape=jax.ShapeDtypeStruct((O, NP), jnp.float32),
  )


def kernel(x, edge_index, edge_weight, W1, b1, W2, b2):
  N, D = x.shape
  H = W1.shape[1]
  O = W2.shape[1]
  E = edge_weight.shape[0]

  BN = 512
  NP = ((N + BN - 1) // BN) * BN
  F = H // NW
  CH = 6400
  SH = max(int(N - 1).bit_length(), 1)
  assert 2 * SH <= 31

  src = edge_index[0]
  dst = edge_index[1]
  xT = jnp.pad(x, ((0, NP - N), (0, 0))).T
  packed = jnp.bitwise_or(jnp.left_shift(dst, SH), src)

  deg_k = _make_deg_kernel(NP, E)
  agg_k = _make_agg_kernel(NP, E, F, CH, SH)
  tc_prep = _make_tc_prep(NP, D, H, BN)
  tc_mid = _make_tc_mid(NP, H, O, BN)

  parts = deg_k(dst, edge_weight)
  y1, dinv = tc_prep(parts, xT, W1)
  dinv_flat = dinv.reshape(NP)
  h = agg_k(y1, packed, edge_weight, dinv_flat, b1)
  y2 = tc_mid(h, dinv, W2)
  outT = agg_k(y2, packed, edge_weight, dinv_flat, b2)
  return outT[:, :N].T


# speedup vs baseline: 1.1368x; 1.0137x over previous
import functools

import jax
import jax.numpy as jnp
from jax import lax
from jax.experimental import pallas as pl
from jax.experimental.pallas import tpu as pltpu
from jax.experimental.pallas import tpu_sc as plsc

LANES = 16
NCORES = 2
NSUB = 16
NW = NCORES * NSUB


def _wid():
  return lax.axis_index("s") * NCORES + lax.axis_index("c")


def _make_deg_kernel(NP, E):
  EPW = E // NW
  mesh = plsc.VectorSubcoreMesh(core_axis_name="c", subcore_axis_name="s",
                                num_cores=NCORES, num_subcores=NSUB)

  @functools.partial(
      pl.kernel,
      out_type=jax.ShapeDtypeStruct((NW, NP), jnp.float32),
      mesh=mesh,
      scratch_types=[
          pltpu.VMEM((EPW,), jnp.int32),
          pltpu.VMEM((EPW,), jnp.float32),
          pltpu.VMEM((NP,), jnp.float32),
      ],
      compiler_params=pltpu.CompilerParams(needs_layout_passes=False),
  )
  def deg_k(dst_hbm, ew_hbm, out_hbm, dst_v, ew_v, acc_v):
    wid = _wid()
    base = wid * EPW
    pltpu.sync_copy(dst_hbm.at[pl.ds(base, EPW)], dst_v)
    pltpu.sync_copy(ew_hbm.at[pl.ds(base, EPW)], ew_v)

    @plsc.parallel_loop(0, NP // LANES, unroll=8)
    def _(i):
      acc_v[pl.ds(i * LANES, LANES)] = jnp.zeros((LANES,), jnp.float32)

    @plsc.parallel_loop(0, EPW // LANES, unroll=8)
    def _(g):
      dv = dst_v[pl.ds(g * LANES, LANES)]
      wv = ew_v[pl.ds(g * LANES, LANES)]
      plsc.addupdate_scatter(acc_v, [dv], wv)

    pltpu.sync_copy(acc_v, out_hbm.at[wid])

  return deg_k


def _make_agg_kernel(NP, E, F, CH, SH):
  mesh = plsc.VectorSubcoreMesh(core_axis_name="c", subcore_axis_name="s",
                                num_cores=NCORES, num_subcores=NSUB)
  NCH = E // CH
  FP = F // 2
  assert NCH % 2 == 0 and CH % LANES == 0 and F % 2 == 0
  scratch = (
      [pltpu.VMEM((NP,), jnp.int32) for _ in range(FP)]
      + [pltpu.VMEM((NP,), jnp.float32) for _ in range(F)]
      + [pltpu.VMEM((2, CH), jnp.int32),
         pltpu.VMEM((2, CH), jnp.float32),
         pltpu.VMEM((NP,), jnp.float32),
         pltpu.VMEM((NW * F,), jnp.float32),
         pltpu.SemaphoreType.DMA,
         pltpu.SemaphoreType.DMA]
  )

  @functools.partial(
      pl.kernel,
      out_type=jax.ShapeDtypeStruct((NW * F, NP), jnp.float32),
      mesh=mesh,
      scratch_types=scratch,
      compiler_params=pltpu.CompilerParams(needs_layout_passes=False),
  )
  def agg_k(y_hbm, pk_hbm, ew_hbm, dinv_hbm, b_hbm, out_hbm, *refs):
    xfp = refs[0:FP]
    acc = refs[FP:FP + F]
    pk_v, ew_v = refs[FP + F:FP + F + 2]
    dinv_v, b_v = refs[FP + F + 2:FP + F + 4]
    sems = refs[FP + F + 4:FP + F + 6]
    wid = _wid()
    rlo = FP * wid
    nhalf = NW * F // 2
    pre = [pltpu.async_copy(y_hbm.at[rlo + p], xfp[p], sems[0])
           for p in range(FP)]
    pre.append(pltpu.async_copy(dinv_hbm, dinv_v, sems[1]))
    pre.append(pltpu.async_copy(b_hbm, b_v, sems[1]))

    @plsc.parallel_loop(0, NP // LANES, unroll=8)
    def _(i):
      for f in range(F):
        acc[f][pl.ds(i * LANES, LANES)] = jnp.zeros((LANES,), jnp.float32)

    for cp in pre:
      cp.wait()

    def issue(c, b):
      base = c * CH
      pltpu.async_copy(pk_hbm.at[pl.ds(base, CH)], pk_v.at[b], sems[b])
      pltpu.async_copy(ew_hbm.at[pl.ds(base, CH)], ew_v.at[b], sems[b])

    def drain(c, b):
      base = c * CH
      pltpu.make_async_copy(pk_hbm.at[pl.ds(base, CH)], pk_v.at[b],
                            sems[b]).wait()
      pltpu.make_async_copy(ew_hbm.at[pl.ds(base, CH)], ew_v.at[b],
                            sems[b]).wait()

    def process(b):
      @plsc.parallel_loop(0, CH // LANES, unroll=8)
      def _(g):
        pv = pk_v[b, pl.ds(g * LANES, LANES)]
        wv = ew_v[b, pl.ds(g * LANES, LANES)]
        sv = pv & ((1 << SH) - 1)
        dv = lax.shift_right_logical(pv, SH)
        for p in range(FP):
          gp = plsc.load_gather(xfp[p], [sv])
          ga, gb = plsc.unpack(plsc.bitcast(gp, jnp.bfloat16),
                               format=plsc.PackFormat.INTERLEAVED,
                               preferred_element_type=jnp.float32)
          plsc.addupdate_scatter(acc[2 * p], [dv], ga * wv)
          plsc.addupdate_scatter(acc[2 * p + 1], [dv], gb * wv)

    issue(0, 0)

    def chunk_pair(c2, carry):
      c = 2 * c2
      issue(c + 1, 1)
      drain(c, 0)
      process(0)

      @pl.when(c2 < NCH // 2 - 1)
      def _():
        issue(c + 2, 0)

      drain(c + 1, 1)
      process(1)
      return carry

    lax.fori_loop(0, NCH // 2, chunk_pair, 0)

    bspl = {}
    for p in range(FP):
      bspl[2 * p] = plsc.load_gather(
          b_v, [jnp.zeros((LANES,), jnp.int32) + (rlo + p)])
      bspl[2 * p + 1] = plsc.load_gather(
          b_v, [jnp.zeros((LANES,), jnp.int32) + (rlo + p + nhalf)])

    @plsc.parallel_loop(0, NP // LANES, unroll=8)
    def _(i):
      sl = pl.ds(i * LANES, LANES)
      dv = dinv_v[sl]
      for p in range(FP):
        ya, yb = plsc.unpack(plsc.bitcast(xfp[p][sl], jnp.bfloat16),
                             format=plsc.PackFormat.INTERLEAVED,
                             preferred_element_type=jnp.float32)
        acc[2 * p][sl] = dv * (acc[2 * p][sl] + ya) + bspl[2 * p]
        acc[2 * p + 1][sl] = dv * (acc[2 * p + 1][sl] + yb) + bspl[2 * p + 1]

    post = []
    for p in range(FP):
      post.append(pltpu.async_copy(acc[2 * p], out_hbm.at[rlo + p], sems[0]))
      post.append(pltpu.async_copy(acc[2 * p + 1],
                                   out_hbm.at[rlo + p + nhalf], sems[0]))
    for cp in post:
      cp.wait()

  return agg_k


def _pack_rows_bf16(y, H):
  u16 = lax.bitcast_convert_type(y.astype(jnp.bfloat16), jnp.uint16)
  top = u16[:H // 2].astype(jnp.uint32)
  bot = u16[H // 2:].astype(jnp.uint32)
  return lax.bitcast_convert_type(top | (bot << 16), jnp.int32)


def _make_tc_prep(NP, D, H, BN):
  def body(parts_ref, xT_ref, w1_ref, y1_ref, dinv_ref):
    deg = 1.0 + jnp.sum(parts_ref[...], axis=0)
    dinv = jnp.where(deg > 0, lax.rsqrt(jnp.maximum(deg, 1e-12)), 0.0)
    xw = lax.dot_general(w1_ref[...], xT_ref[...],
                         (((0,), (0,)), ((), ())),
                         preferred_element_type=jnp.float32)
    y1_ref[...] = _pack_rows_bf16(xw * dinv[None, :], H)
    dinv_ref[...] = dinv[None, :]

  return pl.pallas_call(
      body,
      grid=(NP // BN,),
      in_specs=[
          pl.BlockSpec((NW, BN), lambda j: (0, j)),
          pl.BlockSpec((D, BN), lambda j: (0, j)),
          pl.BlockSpec((D, H), lambda j: (0, 0)),
      ],
      out_specs=[
          pl.BlockSpec((H // 2, BN), lambda j: (0, j)),
          pl.BlockSpec((1, BN), lambda j: (0, j)),
      ],
      out_shape=[
          jax.ShapeDtypeStruct((H // 2, NP), jnp.int32),
          jax.ShapeDtypeStruct((1, NP), jnp.float32),
      ],
  )


def _make_tc_mid(NP, H, O, BN):
  def body(h_ref, dinv_ref, w2_ref, y2_ref):
    xw2 = lax.dot_general(w2_ref[...], h_ref[...],
                          (((0,), (0,)), ((), ())),
                          preferred_element_type=jnp.float32)
    y2_ref[...] = _pack_rows_bf16(xw2 * dinv_ref[...], O)

  return pl.pallas_call(
      body,
      grid=(NP // BN,),
      in_specs=[
          pl.BlockSpec((H, BN), lambda j: (0, j)),
          pl.BlockSpec((1, BN), lambda j: (0, j)),
          pl.BlockSpec((H, O), lambda j: (0, 0)),
      ],
      out_specs=pl.BlockSpec((O // 2, BN), lambda j: (0, j)),
      out_shape=jax.ShapeDtypeStruct((O // 2, NP), jnp.int32),
  )


def kernel(x, edge_index, edge_weight, W1, b1, W2, b2):
  N, D = x.shape
  H = W1.shape[1]
  O = W2.shape[1]
  E = edge_weight.shape[0]

  BN = 512
  NP = ((N + BN - 1) // BN) * BN
  F = H // NW
  CH = 6400
  SH = max(int(N - 1).bit_length(), 1)
  assert 2 * SH <= 31

  src = edge_index[0]
  dst = edge_index[1]
  xT = jnp.pad(x, ((0, NP - N), (0, 0))).T
  packed = jnp.bitwise_or(jnp.left_shift(dst, SH), src)

  deg_k = _make_deg_kernel(NP, E)
  agg_k = _make_agg_kernel(NP, E, F, CH, SH)
  tc_prep = _make_tc_prep(NP, D, H, BN)
  tc_mid = _make_tc_mid(NP, H, O, BN)

  parts = deg_k(dst, edge_weight)
  y1, dinv = tc_prep(parts, xT, W1)
  dinv_flat = dinv.reshape(NP)
  h = agg_k(y1, packed, edge_weight, dinv_flat, b1)
  y2 = tc_mid(h, dinv, W2)
  outT = agg_k(y2, packed, edge_weight, dinv_flat, b2)
  return outT[:, :N].T
